# Initial kernel scaffold; baseline (speedup 1.0000x reference)
import jax
import jax.numpy as jnp
from jax import lax
from jax.experimental import pallas as pl
from jax.experimental.pallas import tpu as pltpu
from jax.experimental.pallas import tpu_sc as plsc

_NC = 2
_NS = 16
_NW = _NC * _NS
_L = 16

_N_WEIGHTS = 100000
_PAD_W = 100352
_B, _T, _F = 4096, 200, 8
_N_ENTRIES = _B * _T
_PER_W = _N_ENTRIES // _NW
_CHUNK = 1024
_N_CHUNKS = _PER_W // _CHUNK
_GROUPS = _CHUNK // _L


def _softmax_body(lw_ref, out_ref):
    v = lw_ref[...]
    m = jnp.max(v)
    e = jnp.exp(v - m)
    out_ref[...] = e * (1.0 / jnp.sum(e))


def _softmax_tc(lw_pad):
    return pl.pallas_call(
        _softmax_body,
        out_shape=jax.ShapeDtypeStruct((_PAD_W // 128, 128), jnp.float32),
    )(lw_pad)


def _gather_body(x_hbm, w_hbm, out_hbm, wtab, xbuf, obuf):
    wid = lax.axis_index("s") * _NC + lax.axis_index("c")
    base = wid * _PER_W

    pltpu.sync_copy(w_hbm, wtab)

    lane8 = lax.iota(jnp.int32, _L) * _F

    def chunk_body(c, _):
        ebase = base + c * _CHUNK
        pltpu.sync_copy(x_hbm.at[pl.ds(ebase * _F, _CHUNK * _F)], xbuf)

        def grp(j, _):
            pos = lane8 + j * (_L * _F)
            vals = plsc.load_gather(xbuf, [pos])
            idxv = vals.astype(jnp.int32)
            g = plsc.load_gather(wtab, [idxv])
            obuf[pl.ds(j * _L, _L)] = g
            return 0

        lax.fori_loop(0, _GROUPS, grp, 0, unroll=4)
        pltpu.sync_copy(obuf, out_hbm.at[pl.ds(ebase, _CHUNK)])
        return 0

    lax.fori_loop(0, _N_CHUNKS, chunk_body, 0)


@jax.jit
def kernel(x, logit_weights):
    lw_pad = jnp.pad(
        logit_weights, (0, _PAD_W - _N_WEIGHTS), constant_values=-jnp.inf
    ).reshape(_PAD_W // 128, 128)
    w = _softmax_tc(lw_pad).reshape(_PAD_W)

    mesh = plsc.VectorSubcoreMesh(core_axis_name="c", subcore_axis_name="s")
    sc = pl.kernel(
        _gather_body,
        out_type=jax.ShapeDtypeStruct((_N_ENTRIES,), jnp.float32),
        mesh=mesh,
        scratch_types=[
            pltpu.VMEM((_PAD_W,), jnp.float32),
            pltpu.VMEM((_CHUNK * _F,), jnp.float32),
            pltpu.VMEM((_CHUNK,), jnp.float32),
        ],
    )
    out_flat = sc(x.reshape(-1), w)
    return out_flat.reshape(_B, _T)


# speedup vs baseline: 23.9646x; 23.9646x over previous
import jax
import jax.numpy as jnp
from jax import lax
from jax.experimental import pallas as pl
from jax.experimental.pallas import tpu as pltpu
from jax.experimental.pallas import tpu_sc as plsc

_NC = 2
_NS = 16
_NW = _NC * _NS
_L = 16

_N_WEIGHTS = 100000
_PAD_W = 100352
_B, _T, _F = 4096, 200, 8
_N_ENTRIES = _B * _T
_PER_W = _N_ENTRIES // _NW
_CHUNK = 1024
_N_CHUNKS = _PER_W // _CHUNK
_GROUPS = _CHUNK // _L


def _softmax_body(lw_ref, out_ref):
    v = lw_ref[...]
    m = jnp.max(v)
    e = jnp.exp(v - m)
    out_ref[...] = e * (1.0 / jnp.sum(e))


def _softmax_tc(lw_pad):
    return pl.pallas_call(
        _softmax_body,
        out_shape=jax.ShapeDtypeStruct((_PAD_W // 128, 128), jnp.float32),
    )(lw_pad)


def _gather_body(x_hbm, w_hbm, out_hbm, wtab, xbuf, obuf):
    wid = lax.axis_index("s") * _NC + lax.axis_index("c")
    base = wid * _PER_W

    pltpu.sync_copy(w_hbm, wtab)

    lane8 = lax.iota(jnp.int32, _L) * _F

    def chunk_body(c, _):
        ebase = base + c * _CHUNK
        pltpu.sync_copy(x_hbm.at[pl.ds(ebase * _F, _CHUNK * _F)], xbuf)

        def grp(j, _):
            pos = lane8 + j * (_L * _F)
            vals = plsc.load_gather(xbuf, [pos])
            idxv = vals.astype(jnp.int32)
            g = plsc.load_gather(wtab, [idxv])
            obuf[pl.ds(j * _L, _L)] = g
            return 0

        lax.fori_loop(0, _GROUPS, grp, 0, unroll=4)
        pltpu.sync_copy(obuf, out_hbm.at[pl.ds(ebase, _CHUNK)])
        return 0

    lax.fori_loop(0, _N_CHUNKS, chunk_body, 0)


@jax.jit
def kernel(x, logit_weights):
    lw_pad = jnp.pad(
        logit_weights, (0, _PAD_W - _N_WEIGHTS), constant_values=-jnp.inf
    ).reshape(_PAD_W // 128, 128)
    w = _softmax_tc(lw_pad).reshape(_PAD_W)

    mesh = plsc.VectorSubcoreMesh(core_axis_name="c", subcore_axis_name="s")
    sc = pl.kernel(
        _gather_body,
        out_type=jax.ShapeDtypeStruct((_N_ENTRIES,), jnp.float32),
        mesh=mesh,
        scratch_types=[
            pltpu.VMEM((_PAD_W,), jnp.float32),
            pltpu.VMEM((_CHUNK * _F,), jnp.float32),
            pltpu.VMEM((_CHUNK,), jnp.float32),
        ],
        compiler_params=pltpu.CompilerParams(needs_layout_passes=False),
    )
    out_flat = sc(x.reshape(-1), w)
    return out_flat.reshape(_B, _T)


# speedup vs baseline: 170.0160x; 7.0945x over previous
import jax
import jax.numpy as jnp
from jax import lax
from jax.experimental import pallas as pl
from jax.experimental.pallas import tpu as pltpu
from jax.experimental.pallas import tpu_sc as plsc

_NC = 2
_NS = 16
_NW = _NC * _NS
_L = 16

_N_WEIGHTS = 100000
_PAD_W = 100352
_B, _T, _F = 4096, 200, 8
_BT = _B // 128
_TI = _T // 8
_GROUPS = 8 * 128 // _L


def _softmax_body(lw_ref, out_ref):
    v = lw_ref[...]
    m = jnp.max(v)
    e = jnp.exp(v - m)
    out_ref[...] = e * (1.0 / jnp.sum(e))


def _softmax_tc(lw_pad):
    return pl.pallas_call(
        _softmax_body,
        out_shape=jax.ShapeDtypeStruct((_PAD_W // 128, 128), jnp.float32),
    )(lw_pad)


def _gather_body(xv_hbm, w_hbm, out_hbm, wtab, xbuf, obuf):
    w = lax.axis_index("c") * _NS + lax.axis_index("s")

    pltpu.sync_copy(w_hbm, wtab)

    def chunk_body(i, _):
        pltpu.sync_copy(xv_hbm.at[pl.ds(i * 8, 8), w, 0, :], xbuf)

        def grp(j, _):
            r = j // 8
            gg = j % 8
            vals = xbuf[r, pl.ds(gg * _L, _L)]
            idxv = vals.astype(jnp.int32)
            g = plsc.load_gather(wtab, [idxv])
            obuf[r, pl.ds(gg * _L, _L)] = g
            return 0

        lax.fori_loop(0, _GROUPS, grp, 0, unroll=8)
        pltpu.sync_copy(obuf, out_hbm.at[i, w, :, :])
        return 0

    lax.fori_loop(0, _TI, chunk_body, 0)


@jax.jit
def kernel(x, logit_weights):
    lw_pad = jnp.pad(
        logit_weights, (0, _PAD_W - _N_WEIGHTS), constant_values=-jnp.inf
    ).reshape(_PAD_W // 128, 128)
    wts = _softmax_tc(lw_pad).reshape(_PAD_W)

    xv = x.transpose(1, 0, 2).reshape(_T, _BT, 128, _F).transpose(0, 1, 3, 2)

    mesh = plsc.VectorSubcoreMesh(core_axis_name="c", subcore_axis_name="s")
    sc = pl.kernel(
        _gather_body,
        out_type=jax.ShapeDtypeStruct((_TI, _BT, 8, 128), jnp.float32),
        mesh=mesh,
        scratch_types=[
            pltpu.VMEM((_PAD_W,), jnp.float32),
            pltpu.VMEM((8, 128), jnp.float32),
            pltpu.VMEM((8, 128), jnp.float32),
        ],
        compiler_params=pltpu.CompilerParams(needs_layout_passes=False),
    )
    res = sc(xv, wts)
    return res.transpose(1, 3, 0, 2).reshape(_B, _T)
